# baseline (speedup 1.0000x reference)
import functools

import jax
import jax.numpy as jnp
from jax import lax
from jax.experimental import pallas as pl
from jax.experimental.pallas import tpu as pltpu
from jax.experimental.pallas import tpu_sc as plsc

_D = 64
_BATCH = 16384
_FIELDS = 26
_TOTAL = _BATCH * _FIELDS
_NC, _NS = 2, 16
_NW = _NC * _NS
_CHUNK = 128
_PER_W = _TOTAL // _NW
_NCHUNK = _PER_W // _CHUNK

_mesh = plsc.VectorSubcoreMesh(
    core_axis_name="c", subcore_axis_name="s",
    num_cores=_NC, num_subcores=_NS,
)


@functools.partial(
    pl.kernel,
    out_type=jax.ShapeDtypeStruct((_TOTAL, _D), jnp.float32),
    mesh=_mesh,
    scratch_types=[
        pltpu.VMEM((_NCHUNK, _CHUNK), jnp.int32),
        pltpu.VMEM((_CHUNK, _D), jnp.float32),
        pltpu.SemaphoreType.DMA,
    ],
    compiler_params=pltpu.CompilerParams(use_tc_tiling_on_sc=False),
)
def _emb_lookup(idx_hbm, table_hbm, out_hbm, idx_v, rows_v, sem):
    wid = lax.axis_index("s") * _NC + lax.axis_index("c")
    cbase = wid * _NCHUNK
    pltpu.sync_copy(idx_hbm.at[pl.ds(cbase, _NCHUNK)], idx_v)

    @pl.loop(0, _NCHUNK)
    def _chunk_loop(j):
        pltpu.async_copy(table_hbm.at[idx_v.at[j]], rows_v, sem).wait()
        pltpu.sync_copy(rows_v, out_hbm.at[pl.ds((cbase + j) * _CHUNK, _CHUNK)])


def kernel(x, table):
    idx = x.reshape(_TOTAL // _CHUNK, _CHUNK).astype(jnp.int32)
    out = _emb_lookup(idx, table)
    return out.reshape(_BATCH, _FIELDS, _D)


# speedup vs baseline: 1.0802x; 1.0802x over previous
import functools

import jax
import jax.numpy as jnp
from jax import lax
from jax.experimental import pallas as pl
from jax.experimental.pallas import tpu as pltpu
from jax.experimental.pallas import tpu_sc as plsc

_D = 64
_BATCH = 16384
_FIELDS = 26
_TOTAL = _BATCH * _FIELDS
_NC, _NS = 2, 16
_NW = _NC * _NS
_CHUNK = 128
_PER_W = _TOTAL // _NW
_NCHUNK = _PER_W // _CHUNK
_GROUP = 4
_GROWS = _GROUP * _CHUNK
_NGROUP = _NCHUNK // _GROUP

_mesh = plsc.VectorSubcoreMesh(
    core_axis_name="c", subcore_axis_name="s",
    num_cores=_NC, num_subcores=_NS,
)


@functools.partial(
    pl.kernel,
    out_type=jax.ShapeDtypeStruct((_TOTAL, _D), jnp.float32),
    mesh=_mesh,
    scratch_types=[
        pltpu.VMEM((_NCHUNK, _CHUNK), jnp.int32),
        pltpu.VMEM((_GROWS, _D), jnp.float32),
        pltpu.VMEM((_GROWS, _D), jnp.float32),
        pltpu.SemaphoreType.DMA,
    ],
    compiler_params=pltpu.CompilerParams(use_tc_tiling_on_sc=False),
)
def _emb_lookup(idx_hbm, table_hbm, out_hbm, idx_v, buf_a, buf_b, gsem):
    wid = lax.axis_index("s") * _NC + lax.axis_index("c")
    cbase = wid * _NCHUNK
    rbase = wid * _PER_W
    pltpu.sync_copy(idx_hbm.at[pl.ds(cbase, _NCHUNK)], idx_v)

    def fire(g, buf):
        for b in range(_GROUP):
            pltpu.async_copy(
                table_hbm.at[idx_v.at[g * _GROUP + b]],
                buf.at[pl.ds(b * _CHUNK, _CHUNK)],
                gsem,
            )

    def drain(buf):
        pltpu.make_async_copy(out_hbm.at[pl.ds(0, _GROWS)], buf, gsem).wait()

    fire(0, buf_a)

    @pl.loop(0, _NGROUP, step=2)
    def _group_loop(g):
        fire(g + 1, buf_b)
        drain(buf_a)
        pltpu.sync_copy(buf_a, out_hbm.at[pl.ds(rbase + g * _GROWS, _GROWS)])

        @pl.when(g + 2 < _NGROUP)
        def _():
            fire(g + 2, buf_a)

        drain(buf_b)
        pltpu.sync_copy(buf_b, out_hbm.at[pl.ds(rbase + (g + 1) * _GROWS, _GROWS)])


def kernel(x, table):
    idx = x.reshape(_TOTAL // _CHUNK, _CHUNK).astype(jnp.int32)
    out = _emb_lookup(idx, table)
    return out.reshape(_BATCH, _FIELDS, _D)
